# baseline (speedup 1.0000x reference)
import functools

import jax
import jax.numpy as jnp
from jax import lax
from jax.experimental import pallas as pl
from jax.experimental.pallas import tpu as pltpu
from jax.experimental.pallas import tpu_sc as plsc

_EMBED_DIM = 64
_BATCH = 4096
_HIST = 50
_NTOT = _BATCH * _HIST

_info = plsc.get_sparse_core_info()
_NC, _NS = _info.num_cores, _info.num_subcores
_NW = _NC * _NS
_B_PER_W = _NTOT // _NW
_CHUNK = 128
_NCHUNK = _B_PER_W // _CHUNK

_mesh = plsc.VectorSubcoreMesh(core_axis_name="c", subcore_axis_name="s")


@functools.partial(
    pl.kernel,
    mesh=_mesh,
    out_type=jax.ShapeDtypeStruct((_NTOT, _EMBED_DIM), jnp.float32),
    scratch_types=[
        pltpu.VMEM((_B_PER_W,), jnp.int32),
        pltpu.VMEM((_CHUNK, _EMBED_DIM), jnp.float32),
        pltpu.SemaphoreType.DMA,
    ],
    compiler_params=pltpu.CompilerParams(use_tc_tiling_on_sc=False),
)
def _gather_kernel(table_hbm, idx_hbm, out_hbm, idx_v, rows_v, sem):
    wid = lax.axis_index("s") * _NC + lax.axis_index("c")
    base = wid * _B_PER_W
    pltpu.sync_copy(idx_hbm.at[pl.ds(base, _B_PER_W)], idx_v)

    def body(c, carry):
        off = pl.multiple_of(c * _CHUNK, 8)
        pltpu.async_copy(
            table_hbm.at[idx_v.at[pl.ds(off, _CHUNK)]], rows_v, sem
        ).wait()
        pltpu.sync_copy(rows_v, out_hbm.at[pl.ds(base + off, _CHUNK)])
        return carry

    lax.fori_loop(0, _NCHUNK, body, 0)


def kernel(input_ids, table):
    idx = input_ids.reshape(_NTOT).astype(jnp.int32)
    out = _gather_kernel(table, idx)
    return out.reshape(_BATCH, _HIST, _EMBED_DIM)


# speedup vs baseline: 1.0413x; 1.0413x over previous
import functools

import jax
import jax.numpy as jnp
from jax import lax
from jax.experimental import pallas as pl
from jax.experimental.pallas import tpu as pltpu
from jax.experimental.pallas import tpu_sc as plsc

_EMBED_DIM = 64
_BATCH = 4096
_HIST = 50
_NTOT = _BATCH * _HIST

_info = plsc.get_sparse_core_info()
_NC, _NS = _info.num_cores, _info.num_subcores
_NW = _NC * _NS
_B_PER_W = _NTOT // _NW
_CHUNK = 128
_NCHUNK = _B_PER_W // _CHUNK
_NBUF = 5
_DEPTH = 2

_mesh = plsc.VectorSubcoreMesh(core_axis_name="c", subcore_axis_name="s")


@functools.partial(
    pl.kernel,
    mesh=_mesh,
    out_type=jax.ShapeDtypeStruct((_NTOT, _EMBED_DIM), jnp.float32),
    scratch_types=[
        pltpu.VMEM((_B_PER_W,), jnp.int32),
        pltpu.VMEM((_NBUF, _CHUNK, _EMBED_DIM), jnp.float32),
        pltpu.SemaphoreType.DMA((_NBUF,)),
        pltpu.SemaphoreType.DMA((_NBUF,)),
    ],
    compiler_params=pltpu.CompilerParams(use_tc_tiling_on_sc=False),
)
def _gather_kernel(table_hbm, idx_hbm, out_hbm, idx_v, rows_v, gsem, osem):
    wid = lax.axis_index("s") * _NC + lax.axis_index("c")
    base = wid * _B_PER_W
    pltpu.sync_copy(idx_hbm.at[pl.ds(base, _B_PER_W)], idx_v)

    def gather(g, b):
        off = pl.multiple_of(g * _CHUNK, 8)
        return pltpu.make_async_copy(
            table_hbm.at[idx_v.at[pl.ds(off, _CHUNK)]], rows_v.at[b], gsem.at[b]
        )

    def scatter(g, b):
        off = pl.multiple_of(g * _CHUNK, 8)
        return pltpu.make_async_copy(
            rows_v.at[b], out_hbm.at[pl.ds(base + off, _CHUNK)], osem.at[b]
        )

    for g in range(_DEPTH):
        gather(g, g).start()

    for g in range(_NBUF - _DEPTH):
        gather(g, g % _NBUF).wait()
        scatter(g, g % _NBUF).start()
        gather(g + _DEPTH, (g + _DEPTH) % _NBUF).start()

    _G0 = _NBUF - _DEPTH
    _NSTEADY = (_NCHUNK - _DEPTH) - _G0

    @pl.loop(0, _NSTEADY // _NBUF)
    def _steady(go):
        for db in range(_NBUF):
            g = _G0 + go * _NBUF + db
            b = (_G0 + db) % _NBUF
            gather(g, b).wait()
            scatter(g, b).start()
            f = g + _DEPTH
            bf = (_G0 + db + _DEPTH) % _NBUF
            scatter(f - _NBUF, bf).wait()
            gather(f, bf).start()

    for g in range(_NCHUNK - _DEPTH, _NCHUNK):
        b = g % _NBUF
        gather(g, b).wait()
        scatter(g, b).start()

    for g in range(_NCHUNK - _NBUF, _NCHUNK):
        scatter(g, g % _NBUF).wait()


def kernel(input_ids, table):
    idx = input_ids.reshape(_NTOT).astype(jnp.int32)
    out = _gather_kernel(table, idx)
    return out.reshape(_BATCH, _HIST, _EMBED_DIM)


# speedup vs baseline: 1.0471x; 1.0055x over previous
import functools

import jax
import jax.numpy as jnp
from jax import lax
from jax.experimental import pallas as pl
from jax.experimental.pallas import tpu as pltpu
from jax.experimental.pallas import tpu_sc as plsc

_EMBED_DIM = 64
_BATCH = 4096
_HIST = 50
_NTOT = _BATCH * _HIST

_info = plsc.get_sparse_core_info()
_NC, _NS = _info.num_cores, _info.num_subcores
_NW = _NC * _NS
_B_PER_W = _NTOT // _NW
_CHUNK = 256
_NCHUNK = _B_PER_W // _CHUNK
_NBUF = 5
_DEPTH = 2

_mesh = plsc.VectorSubcoreMesh(core_axis_name="c", subcore_axis_name="s")


@functools.partial(
    pl.kernel,
    mesh=_mesh,
    out_type=jax.ShapeDtypeStruct((_NTOT, _EMBED_DIM), jnp.float32),
    scratch_types=[
        pltpu.VMEM((_B_PER_W,), jnp.int32),
        pltpu.VMEM((_NBUF, _CHUNK, _EMBED_DIM), jnp.float32),
        pltpu.SemaphoreType.DMA((_NBUF,)),
        pltpu.SemaphoreType.DMA((_NBUF,)),
    ],
    compiler_params=pltpu.CompilerParams(use_tc_tiling_on_sc=False),
)
def _gather_kernel(table_hbm, idx_hbm, out_hbm, idx_v, rows_v, gsem, osem):
    wid = lax.axis_index("s") * _NC + lax.axis_index("c")
    base = wid * _B_PER_W
    pltpu.sync_copy(idx_hbm.at[pl.ds(base, _B_PER_W)], idx_v)

    def gather(g, b):
        off = pl.multiple_of(g * _CHUNK, 8)
        return pltpu.make_async_copy(
            table_hbm.at[idx_v.at[pl.ds(off, _CHUNK)]], rows_v.at[b], gsem.at[b]
        )

    def scatter(g, b):
        off = pl.multiple_of(g * _CHUNK, 8)
        return pltpu.make_async_copy(
            rows_v.at[b], out_hbm.at[pl.ds(base + off, _CHUNK)], osem.at[b]
        )

    for g in range(_DEPTH):
        gather(g, g).start()

    for g in range(_NBUF - _DEPTH):
        gather(g, g % _NBUF).wait()
        scatter(g, g % _NBUF).start()
        gather(g + _DEPTH, (g + _DEPTH) % _NBUF).start()

    _G0 = _NBUF - _DEPTH
    _NSTEADY = (_NCHUNK - _DEPTH) - _G0

    @pl.loop(0, _NSTEADY // _NBUF)
    def _steady(go):
        for db in range(_NBUF):
            g = _G0 + go * _NBUF + db
            b = (_G0 + db) % _NBUF
            gather(g, b).wait()
            scatter(g, b).start()
            f = g + _DEPTH
            bf = (_G0 + db + _DEPTH) % _NBUF
            scatter(f - _NBUF, bf).wait()
            gather(f, bf).start()

    for g in range(_NCHUNK - _DEPTH, _NCHUNK):
        b = g % _NBUF
        gather(g, b).wait()
        scatter(g, b).start()

    for g in range(_NCHUNK - _NBUF, _NCHUNK):
        scatter(g, g % _NBUF).wait()


def kernel(input_ids, table):
    idx = input_ids.reshape(_NTOT).astype(jnp.int32)
    out = _gather_kernel(table, idx)
    return out.reshape(_BATCH, _HIST, _EMBED_DIM)
